# baseline (device time: 15805 ns/iter reference)
import jax
import jax.numpy as jnp
from jax import lax
from jax.experimental import pallas as pl
from jax.experimental.pallas import tpu as pltpu

K = 16
SUB = 32
G = 4


def kernel(x):
    m, n = x.shape
    lanes = n // SUB
    mh = m // 2

    def body(x_hbm, out_ref, xv, cand_ref, load_sem,
             send_sem_y, recv_sem_y, send_sem_x, recv_sem_x):
        my_x = lax.axis_index("x")
        my_y = lax.axis_index("y")
        nbr_y = (my_x, 1 - my_y)
        nbr_x = (1 - my_x, my_y)
        row0 = my_x * mh

        barrier_sem = pltpu.get_barrier_semaphore()
        for nbr in (nbr_y, nbr_x):
            pl.semaphore_signal(
                barrier_sem, inc=1, device_id=nbr,
                device_id_type=pl.DeviceIdType.MESH,
            )

        load = pltpu.make_async_copy(
            x_hbm.at[pl.ds(row0, mh), :], xv, load_sem)
        load.start()
        load.wait()

        neg = jnp.float32(-jnp.inf)

        gms = []
        for g in range(SUB // G):
            gm = xv[:, (g * G) * lanes:(g * G + 1) * lanes]
            for j in range(g * G + 1, (g + 1) * G):
                gm = jnp.maximum(gm, xv[:, j * lanes:(j + 1) * lanes])
            gms.append(gm)
        m1 = gms[0]
        m2 = jnp.full((mh, lanes), neg, jnp.float32)
        for gm in gms[1:]:
            m2 = jnp.maximum(m2, jnp.minimum(m1, gm))
            m1 = jnp.maximum(m1, gm)

        work, nxt = m1, m2
        cols = []
        for j in range(K):
            mx = jnp.max(work, axis=1, keepdims=True)
            cols.append(mx)
            if j < K - 1:
                hit = work == mx
                work = jnp.where(hit, nxt, work)
                nxt = jnp.where(hit, neg, nxt)
        cand_ref[0, :, :] = jnp.concatenate(cols, axis=1)

        pl.semaphore_wait(barrier_sem, 2)
        rdma_y = pltpu.make_async_remote_copy(
            src_ref=cand_ref.at[0],
            dst_ref=cand_ref.at[1],
            send_sem=send_sem_y,
            recv_sem=recv_sem_y,
            device_id=nbr_y,
            device_id_type=pl.DeviceIdType.MESH,
        )
        rdma_y.start()
        rdma_y.wait()

        cc = jnp.concatenate([cand_ref[0, :, :], cand_ref[1, :, :]], axis=1)
        outs = []
        for j in range(K):
            mx = jnp.max(cc, axis=1, keepdims=True)
            outs.append(mx)
            if j < K - 1:
                cc = jnp.where(cc == mx, neg, cc)
        out_ref[pl.ds(row0, mh), :] = jnp.concatenate(outs, axis=1)

        rdma_x = pltpu.make_async_remote_copy(
            src_ref=out_ref.at[pl.ds(row0, mh), :],
            dst_ref=out_ref.at[pl.ds(row0, mh), :],
            send_sem=send_sem_x,
            recv_sem=recv_sem_x,
            device_id=nbr_x,
            device_id_type=pl.DeviceIdType.MESH,
        )
        rdma_x.start()
        rdma_x.wait()

    return pl.pallas_call(
        body,
        out_shape=jax.ShapeDtypeStruct((m, K), jnp.float32),
        in_specs=[pl.BlockSpec(memory_space=pl.ANY)],
        out_specs=pl.BlockSpec(memory_space=pltpu.VMEM),
        scratch_shapes=[
            pltpu.VMEM((mh, n), jnp.float32),
            pltpu.VMEM((2, mh, K), jnp.float32),
            pltpu.SemaphoreType.DMA,
            pltpu.SemaphoreType.DMA,
            pltpu.SemaphoreType.DMA,
            pltpu.SemaphoreType.DMA,
            pltpu.SemaphoreType.DMA,
        ],
        compiler_params=pltpu.CompilerParams(collective_id=0),
    )(x)


# device time: 14663 ns/iter; 1.0779x vs baseline; 1.0779x over previous
import jax
import jax.numpy as jnp
from jax import lax
from jax.experimental import pallas as pl
from jax.experimental.pallas import tpu as pltpu

K = 16
SUB = 32
G = 4


def _topk_merge(a, b, k=K):
    neg = jnp.float32(-jnp.inf)
    cc = jnp.concatenate([a, b], axis=1)
    outs = []
    for j in range(k):
        mx = jnp.max(cc, axis=1, keepdims=True)
        outs.append(mx)
        if j < k - 1:
            cc = jnp.where(cc == mx, neg, cc)
    return jnp.concatenate(outs, axis=1)


def kernel(x):
    m, n = x.shape
    lanes = n // SUB
    mh = m // 2

    def body(x_hbm, out_ref, xv, cand_ref, load_sem, send_sems, recv_sems):
        my_x = lax.axis_index("x")
        my_y = lax.axis_index("y")
        peers = [
            (my_x, 1 - my_y),
            (1 - my_x, my_y),
            (1 - my_x, 1 - my_y),
        ]

        barrier_sem = pltpu.get_barrier_semaphore()
        for nbr in peers:
            pl.semaphore_signal(
                barrier_sem, inc=1, device_id=nbr,
                device_id_type=pl.DeviceIdType.MESH,
            )

        row0 = my_x * mh
        load = pltpu.make_async_copy(
            x_hbm.at[pl.ds(row0, mh), :], xv, load_sem)
        load.start()
        load.wait()

        neg = jnp.float32(-jnp.inf)

        gms = []
        for g in range(SUB // G):
            gm = xv[:, (g * G) * lanes:(g * G + 1) * lanes]
            for j in range(g * G + 1, (g + 1) * G):
                gm = jnp.maximum(gm, xv[:, j * lanes:(j + 1) * lanes])
            gms.append(gm)
        m1 = gms[0]
        m2 = jnp.full((mh, lanes), neg, jnp.float32)
        for gm in gms[1:]:
            m2 = jnp.maximum(m2, jnp.minimum(m1, gm))
            m1 = jnp.maximum(m1, gm)

        work, nxt = m1, m2
        cols = []
        for j in range(K):
            mx = jnp.max(work, axis=1, keepdims=True)
            cols.append(mx)
            if j < K - 1:
                hit = work == mx
                work = jnp.where(hit, nxt, work)
                nxt = jnp.where(hit, neg, nxt)
        cand_ref[0, :, :] = jnp.concatenate(cols, axis=1)

        pl.semaphore_wait(barrier_sem, len(peers))
        rdmas = []
        for i, nbr in enumerate(peers):
            rdma = pltpu.make_async_remote_copy(
                src_ref=cand_ref.at[0],
                dst_ref=cand_ref.at[i + 1],
                send_sem=send_sems.at[i],
                recv_sem=recv_sems.at[i],
                device_id=nbr,
                device_id_type=pl.DeviceIdType.MESH,
            )
            rdma.start()
            rdmas.append(rdma)

        rdmas[0].wait_recv()
        out_ref[pl.ds(row0, mh), :] = _topk_merge(
            cand_ref[0, :, :], cand_ref[1, :, :])

        rdmas[1].wait_recv()
        rdmas[2].wait_recv()
        out_ref[pl.ds((1 - my_x) * mh, mh), :] = _topk_merge(
            cand_ref[2, :, :], cand_ref[3, :, :])

        for rdma in rdmas:
            rdma.wait_send()

    return pl.pallas_call(
        body,
        out_shape=jax.ShapeDtypeStruct((m, K), jnp.float32),
        in_specs=[pl.BlockSpec(memory_space=pl.ANY)],
        out_specs=pl.BlockSpec(memory_space=pltpu.VMEM),
        scratch_shapes=[
            pltpu.VMEM((mh, n), jnp.float32),
            pltpu.VMEM((4, mh, K), jnp.float32),
            pltpu.SemaphoreType.DMA,
            pltpu.SemaphoreType.DMA((3,)),
            pltpu.SemaphoreType.DMA((3,)),
        ],
        compiler_params=pltpu.CompilerParams(collective_id=0),
    )(x)


# device time: 14410 ns/iter; 1.0968x vs baseline; 1.0176x over previous
import jax
import jax.numpy as jnp
from jax import lax
from jax.experimental import pallas as pl
from jax.experimental.pallas import tpu as pltpu

K = 16
SUB = 32
G = 4


def _topk_merge(a, b, k=K):
    neg = jnp.float32(-jnp.inf)
    cc = jnp.concatenate([a, b], axis=1)
    outs = []
    for j in range(k):
        mx = jnp.max(cc, axis=1, keepdims=True)
        outs.append(mx)
        if j < k - 1:
            cc = jnp.where(cc == mx, neg, cc)
    return jnp.concatenate(outs, axis=1)


def kernel(x):
    m, n = x.shape
    lanes = n // SUB
    mh = m // 2

    def body(x_ref, out_ref, cand_ref, send_sems, recv_sems):
        my_x = lax.axis_index("x")
        my_y = lax.axis_index("y")
        peers = [
            (my_x, 1 - my_y),
            (1 - my_x, my_y),
            (1 - my_x, 1 - my_y),
        ]

        barrier_sem = pltpu.get_barrier_semaphore()
        for nbr in peers:
            pl.semaphore_signal(
                barrier_sem, inc=1, device_id=nbr,
                device_id_type=pl.DeviceIdType.MESH,
            )

        row0 = my_x * mh
        neg = jnp.float32(-jnp.inf)

        gms = []
        for g in range(SUB // G):
            gm = x_ref[pl.ds(row0, mh), (g * G) * lanes:(g * G + 1) * lanes]
            for j in range(g * G + 1, (g + 1) * G):
                gm = jnp.maximum(
                    gm, x_ref[pl.ds(row0, mh), j * lanes:(j + 1) * lanes])
            gms.append(gm)
        m1 = gms[0]
        m2 = jnp.full((mh, lanes), neg, jnp.float32)
        for gm in gms[1:]:
            m2 = jnp.maximum(m2, jnp.minimum(m1, gm))
            m1 = jnp.maximum(m1, gm)

        work, nxt = m1, m2
        cols = []
        for j in range(K):
            mx = jnp.max(work, axis=1, keepdims=True)
            cols.append(mx)
            if j < K - 1:
                hit = work == mx
                work = jnp.where(hit, nxt, work)
                nxt = jnp.where(hit, neg, nxt)
        cand_ref[0, :, :] = jnp.concatenate(cols, axis=1)

        pl.semaphore_wait(barrier_sem, len(peers))
        rdmas = []
        for i, nbr in enumerate(peers):
            rdma = pltpu.make_async_remote_copy(
                src_ref=cand_ref.at[0],
                dst_ref=cand_ref.at[i + 1],
                send_sem=send_sems.at[i],
                recv_sem=recv_sems.at[i],
                device_id=nbr,
                device_id_type=pl.DeviceIdType.MESH,
            )
            rdma.start()
            rdmas.append(rdma)

        rdmas[0].wait_recv()
        out_ref[pl.ds(row0, mh), :] = _topk_merge(
            cand_ref[0, :, :], cand_ref[1, :, :])

        rdmas[1].wait_recv()
        rdmas[2].wait_recv()
        out_ref[pl.ds((1 - my_x) * mh, mh), :] = _topk_merge(
            cand_ref[2, :, :], cand_ref[3, :, :])

        for rdma in rdmas:
            rdma.wait_send()

    return pl.pallas_call(
        body,
        out_shape=jax.ShapeDtypeStruct((m, K), jnp.float32),
        in_specs=[pl.BlockSpec(memory_space=pltpu.VMEM)],
        out_specs=pl.BlockSpec(memory_space=pltpu.VMEM),
        scratch_shapes=[
            pltpu.VMEM((4, mh, K), jnp.float32),
            pltpu.SemaphoreType.DMA((3,)),
            pltpu.SemaphoreType.DMA((3,)),
        ],
        compiler_params=pltpu.CompilerParams(collective_id=0),
    )(x)
